# baseline (speedup 1.0000x reference)
import functools

import jax
import jax.numpy as jnp
from jax import lax
from jax.experimental import pallas as pl
from jax.experimental.pallas import tpu as pltpu
from jax.experimental.pallas import tpu_sc as plsc

BATCH = 16384
DIM = 64
WIDE = 128
LANES = 16
NUM_WORKERS = 32
B_PER_W = BATCH // NUM_WORKERS
CHUNK = 128
N_CHUNKS = B_PER_W // CHUNK
GROUPS_PER_CHUNK = CHUNK // LANES


def _body(ent_hbm, rel_hbm, hidx_hbm, ridx_hbm, tidx_hbm, out_hbm,
          hidx_v, ridx_v, tidx_v, hhalf_v, rhalf_v, thalf_v,
          hrows_v, rrows_v, trows_v, out_v, *sems):
    wid = lax.axis_index("s") * 2 + lax.axis_index("c")
    row0 = wid * N_CHUNKS
    base = wid * B_PER_W

    pltpu.sync_copy(hidx_hbm.at[pl.ds(row0, N_CHUNKS)], hidx_v)
    pltpu.sync_copy(ridx_hbm.at[pl.ds(row0, N_CHUNKS)], ridx_v)
    pltpu.sync_copy(tidx_hbm.at[pl.ds(row0, N_CHUNKS)], tidx_v)

    m511 = jnp.full((LANES,), 511, jnp.int32)
    for src, dst in ((hidx_v, hhalf_v), (ridx_v, rhalf_v), (tidx_v, thalf_v)):
        for k in range(N_CHUNKS):
            for v in range(CHUNK // LANES):
                sl = pl.ds(v * LANES, LANES)
                e = src[k, sl]
                dst[k, sl] = (
                    lax.shift_left(lax.shift_right_logical(e, 10), 9)
                    + (e & m511))

    lanes = lax.iota(jnp.int32, LANES)
    zero = jnp.zeros((LANES,), jnp.float32)
    half = jnp.full((LANES,), 0.5, jnp.float32)
    three_half = jnp.full((LANES,), 1.5, jnp.float32)
    magic = jnp.full((LANES,), 0x5F3759DF, jnp.int32)
    six = jnp.int32(6)
    one = jnp.int32(1)

    def fire(k, slot):
        s = sems[slot]
        return (
            pltpu.async_copy(ent_hbm.at[hhalf_v.at[k]], hrows_v.at[slot], s),
            pltpu.async_copy(rel_hbm.at[rhalf_v.at[k]], rrows_v.at[slot], s),
            pltpu.async_copy(ent_hbm.at[thalf_v.at[k]], trows_v.at[slot], s),
        )

    def make_group(k, slot):
        def group(r, _):
            sl16 = pl.ds(r * LANES, LANES)
            hv = hidx_v[k, sl16]
            rv = ridx_v[k, sl16]
            tv = tidx_v[k, sl16]
            tot = zero
            for t in range(LANES):
                i = r * LANES + t
                ho = lax.shift_left(lax.shift_right_logical(hv[t], 9) & one, six)
                ro = lax.shift_left(lax.shift_right_logical(rv[t], 9) & one, six)
                to = lax.shift_left(lax.shift_right_logical(tv[t], 9) & one, six)
                acc = zero
                for j in range(DIM // LANES):
                    o = j * LANES
                    d = (hrows_v[slot, i, pl.ds(ho + o, LANES)]
                         + rrows_v[slot, i, pl.ds(ro + o, LANES)]
                         - trows_v[slot, i, pl.ds(to + o, LANES)])
                    acc = acc + d * d
                s = acc[0]
                for c in range(1, LANES):
                    s = s + acc[c]
                tot = jnp.where(lanes == t, s, tot)
            xi = lax.bitcast_convert_type(tot, jnp.int32)
            y = lax.bitcast_convert_type(
                magic - lax.shift_right_logical(xi, 1), jnp.float32)
            hx = half * tot
            for _ in range(3):
                y = y * (three_half - hx * y * y)
            out_v[pl.ds((k * GROUPS_PER_CHUNK + r) * LANES, LANES)] = tot * y
            return 0
        return group

    pending = fire(0, 0)
    for k in range(N_CHUNKS):
        nxt = fire(k + 1, (k + 1) % 2) if k + 1 < N_CHUNKS else None
        for c in pending:
            c.wait()
        lax.fori_loop(0, GROUPS_PER_CHUNK, make_group(k, k % 2), 0)
        pending = nxt

    pltpu.sync_copy(out_v, out_hbm.at[pl.ds(base, B_PER_W)])


TPOSE_C = 1024


def _tpose_body(in_ref, out_ref):
    out_ref[:, 0:DIM] = jnp.transpose(in_ref[:, 0:512], (1, 0))
    out_ref[:, DIM:WIDE] = jnp.transpose(in_ref[:, 512:1024], (1, 0))


def _tc_transpose(table_t, num_rows):
    grid = (num_rows + TPOSE_C - 1) // TPOSE_C
    return pl.pallas_call(
        _tpose_body,
        grid=(grid,),
        in_specs=[pl.BlockSpec((DIM, TPOSE_C), lambda i: (0, i))],
        out_specs=pl.BlockSpec((TPOSE_C // 2, WIDE), lambda i: (i, 0)),
        out_shape=jax.ShapeDtypeStruct(
            (grid * (TPOSE_C // 2), WIDE), jnp.float32),
    )(table_t)


@jax.jit
def kernel(triples, entity_table, relation_table):
    hidx = triples[:, 0].reshape(BATCH // CHUNK, CHUNK)
    ridx = triples[:, 1].reshape(BATCH // CHUNK, CHUNK)
    tidx = triples[:, 2].reshape(BATCH // CHUNK, CHUNK)
    ent2 = _tc_transpose(entity_table.T, entity_table.shape[0])
    rel2 = _tc_transpose(relation_table.T, relation_table.shape[0])

    run = functools.partial(
        pl.kernel,
        out_type=jax.ShapeDtypeStruct((BATCH,), jnp.float32),
        mesh=plsc.VectorSubcoreMesh(core_axis_name="c", subcore_axis_name="s"),
        scratch_types=[
            pltpu.VMEM((N_CHUNKS, CHUNK), jnp.int32),
            pltpu.VMEM((N_CHUNKS, CHUNK), jnp.int32),
            pltpu.VMEM((N_CHUNKS, CHUNK), jnp.int32),
            pltpu.VMEM((N_CHUNKS, CHUNK), jnp.int32),
            pltpu.VMEM((N_CHUNKS, CHUNK), jnp.int32),
            pltpu.VMEM((N_CHUNKS, CHUNK), jnp.int32),
            pltpu.VMEM((2, CHUNK, WIDE), jnp.float32),
            pltpu.VMEM((2, CHUNK, WIDE), jnp.float32),
            pltpu.VMEM((2, CHUNK, WIDE), jnp.float32),
            pltpu.VMEM((B_PER_W,), jnp.float32),
            pltpu.SemaphoreType.DMA,
            pltpu.SemaphoreType.DMA,
        ],
    )(_body)
    return run(ent2, rel2, hidx, ridx, tidx)


# speedup vs baseline: 1.7985x; 1.7985x over previous
import functools

import jax
import jax.numpy as jnp
from jax import lax
from jax.experimental import pallas as pl
from jax.experimental.pallas import tpu as pltpu
from jax.experimental.pallas import tpu_sc as plsc

BATCH = 16384
DIM = 64
ROWS_PER_TILE = 8
LANES = 16
NUM_WORKERS = 32
B_PER_W = BATCH // NUM_WORKERS
IDX_CHUNK = 128
N_IDX_CHUNKS = B_PER_W // IDX_CHUNK
CTRIP = 16
N_CHUNKS = B_PER_W // CTRIP


def _body(ent_hbm, rel_hbm, hidx_hbm, ridx_hbm, tidx_hbm, out_hbm,
          hidx_v, ridx_v, tidx_v, hbuf_v, rbuf_v, tbuf_v, out_v,
          sem0, sem1):
    wid = lax.axis_index("s") * 2 + lax.axis_index("c")
    row0 = wid * N_IDX_CHUNKS
    base = wid * B_PER_W

    pltpu.sync_copy(hidx_hbm.at[pl.ds(row0, N_IDX_CHUNKS)], hidx_v)
    pltpu.sync_copy(ridx_hbm.at[pl.ds(row0, N_IDX_CHUNKS)], ridx_v)
    pltpu.sync_copy(tidx_hbm.at[pl.ds(row0, N_IDX_CHUNKS)], tidx_v)

    sems = (sem0, sem1)
    lanes = lax.iota(jnp.int32, LANES)
    zero = jnp.zeros((LANES,), jnp.float32)
    half = jnp.full((LANES,), 0.5, jnp.float32)
    three_half = jnp.full((LANES,), 1.5, jnp.float32)
    magic = jnp.full((LANES,), 0x5F3759DF, jnp.int32)
    seven = jnp.int32(7)

    def idx_vecs(c):
        k = c // (IDX_CHUNK // CTRIP)
        o = (c % (IDX_CHUNK // CTRIP)) * CTRIP
        sl = pl.ds(o, LANES)
        return hidx_v[k, sl], ridx_v[k, sl], tidx_v[k, sl]

    def tile_slice(e):
        e0 = pl.multiple_of(
            lax.shift_left(lax.shift_right_logical(e, 3), 3), ROWS_PER_TILE)
        return pl.ds(e0, ROWS_PER_TILE)

    def fire(c, slot):
        hv, rv, tv = idx_vecs(c)
        for t in range(LANES):
            pltpu.async_copy(
                ent_hbm.at[tile_slice(hv[t])], hbuf_v.at[slot, t],
                sems[slot])
            pltpu.async_copy(
                rel_hbm.at[tile_slice(rv[t])], rbuf_v.at[slot, t],
                sems[slot])
            pltpu.async_copy(
                ent_hbm.at[tile_slice(tv[t])], tbuf_v.at[slot, t],
                sems[slot])

    def drain(slot):
        for t in range(LANES):
            pltpu.make_async_copy(
                ent_hbm.at[pl.ds(0, ROWS_PER_TILE)], hbuf_v.at[slot, t],
                sems[slot]).wait()
            pltpu.make_async_copy(
                rel_hbm.at[pl.ds(0, ROWS_PER_TILE)], rbuf_v.at[slot, t],
                sems[slot]).wait()
            pltpu.make_async_copy(
                ent_hbm.at[pl.ds(0, ROWS_PER_TILE)], tbuf_v.at[slot, t],
                sems[slot]).wait()

    def compute(c, slot):
        hv, rv, tv = idx_vecs(c)
        tot = zero
        for t in range(LANES):
            hr = hv[t] & seven
            rr = rv[t] & seven
            tr = tv[t] & seven
            acc = zero
            for j in range(DIM // LANES):
                sl = pl.ds(j * LANES, LANES)
                d = (hbuf_v[slot, t, hr, sl] + rbuf_v[slot, t, rr, sl]
                     - tbuf_v[slot, t, tr, sl])
                acc = acc + d * d
            s = acc[0]
            for cc in range(1, LANES):
                s = s + acc[cc]
            tot = jnp.where(lanes == t, s, tot)
        xi = lax.bitcast_convert_type(tot, jnp.int32)
        y = lax.bitcast_convert_type(
            magic - lax.shift_right_logical(xi, 1), jnp.float32)
        hx = half * tot
        for _ in range(3):
            y = y * (three_half - hx * y * y)
        out_v[pl.ds(c * CTRIP, LANES)] = tot * y

    fire(jnp.int32(0), 0)

    def pair(p, _):
        c0 = p * 2
        fire(c0 + 1, 1)
        drain(0)
        compute(c0, 0)
        fire(c0 + 2, 0)
        drain(1)
        compute(c0 + 1, 1)
        return 0

    lax.fori_loop(0, N_CHUNKS // 2 - 1, pair, 0)
    last = jnp.int32(N_CHUNKS - 2)
    fire(last + 1, 1)
    drain(0)
    compute(last, 0)
    drain(1)
    compute(last + 1, 1)

    pltpu.sync_copy(out_v, out_hbm.at[pl.ds(base, B_PER_W)])


@jax.jit
def kernel(triples, entity_table, relation_table):
    hidx = triples[:, 0].reshape(BATCH // IDX_CHUNK, IDX_CHUNK)
    ridx = triples[:, 1].reshape(BATCH // IDX_CHUNK, IDX_CHUNK)
    tidx = triples[:, 2].reshape(BATCH // IDX_CHUNK, IDX_CHUNK)

    run = functools.partial(
        pl.kernel,
        out_type=jax.ShapeDtypeStruct((BATCH,), jnp.float32),
        mesh=plsc.VectorSubcoreMesh(core_axis_name="c", subcore_axis_name="s"),
        scratch_types=[
            pltpu.VMEM((N_IDX_CHUNKS, IDX_CHUNK), jnp.int32),
            pltpu.VMEM((N_IDX_CHUNKS, IDX_CHUNK), jnp.int32),
            pltpu.VMEM((N_IDX_CHUNKS, IDX_CHUNK), jnp.int32),
            pltpu.VMEM((2, CTRIP, ROWS_PER_TILE, DIM), jnp.float32),
            pltpu.VMEM((2, CTRIP, ROWS_PER_TILE, DIM), jnp.float32),
            pltpu.VMEM((2, CTRIP, ROWS_PER_TILE, DIM), jnp.float32),
            pltpu.VMEM((B_PER_W,), jnp.float32),
            pltpu.SemaphoreType.DMA,
            pltpu.SemaphoreType.DMA,
        ],
    )(_body)
    return run(entity_table, relation_table, hidx, ridx, tidx)


# speedup vs baseline: 2.5271x; 1.4051x over previous
import functools

import jax
import jax.numpy as jnp
from jax import lax
from jax.experimental import pallas as pl
from jax.experimental.pallas import tpu as pltpu
from jax.experimental.pallas import tpu_sc as plsc

BATCH = 16384
DIM = 64
WIDE = 128
LANES = 16
NUM_WORKERS = 32
B_PER_W = BATCH // NUM_WORKERS
CHUNK = 128
N_CHUNKS = B_PER_W // CHUNK
GROUPS_PER_CHUNK = CHUNK // LANES


def _body(ent_hbm, rel_hbm, hidx_hbm, ridx_hbm, tidx_hbm, out_hbm,
          hidx_v, ridx_v, tidx_v, hhalf_v, rhalf_v, thalf_v,
          hrows_v, rrows_v, trows_v, out_v, *sems):
    wid = lax.axis_index("s") * 2 + lax.axis_index("c")
    row0 = wid * N_CHUNKS
    base = wid * B_PER_W

    pltpu.sync_copy(hidx_hbm.at[pl.ds(row0, N_CHUNKS)], hidx_v)
    pltpu.sync_copy(ridx_hbm.at[pl.ds(row0, N_CHUNKS)], ridx_v)
    pltpu.sync_copy(tidx_hbm.at[pl.ds(row0, N_CHUNKS)], tidx_v)

    m511 = jnp.full((LANES,), 511, jnp.int32)
    for src, dst in ((hidx_v, hhalf_v), (ridx_v, rhalf_v), (tidx_v, thalf_v)):
        for k in range(N_CHUNKS):
            for v in range(CHUNK // LANES):
                sl = pl.ds(v * LANES, LANES)
                e = src[k, sl]
                dst[k, sl] = (
                    lax.shift_left(lax.shift_right_logical(e, 10), 9)
                    + (e & m511))

    lanes = lax.iota(jnp.int32, LANES)
    zero = jnp.zeros((LANES,), jnp.float32)
    half = jnp.full((LANES,), 0.5, jnp.float32)
    three_half = jnp.full((LANES,), 1.5, jnp.float32)
    magic = jnp.full((LANES,), 0x5F3759DF, jnp.int32)
    six = jnp.int32(6)
    one = jnp.int32(1)

    def fire(k, slot):
        s = sems[slot]
        return (
            pltpu.async_copy(ent_hbm.at[hhalf_v.at[k]], hrows_v.at[slot], s),
            pltpu.async_copy(rel_hbm.at[rhalf_v.at[k]], rrows_v.at[slot], s),
            pltpu.async_copy(ent_hbm.at[thalf_v.at[k]], trows_v.at[slot], s),
        )

    def make_group(k, slot):
        def group(r, _):
            sl16 = pl.ds(r * LANES, LANES)
            hv = hidx_v[k, sl16]
            rv = ridx_v[k, sl16]
            tv = tidx_v[k, sl16]
            tot = zero
            for t in range(LANES):
                i = r * LANES + t
                ho = lax.shift_left(lax.shift_right_logical(hv[t], 9) & one, six)
                ro = lax.shift_left(lax.shift_right_logical(rv[t], 9) & one, six)
                to = lax.shift_left(lax.shift_right_logical(tv[t], 9) & one, six)
                acc = zero
                for j in range(DIM // LANES):
                    o = j * LANES
                    d = (hrows_v[slot, i, pl.ds(ho + o, LANES)]
                         + rrows_v[slot, i, pl.ds(ro + o, LANES)]
                         - trows_v[slot, i, pl.ds(to + o, LANES)])
                    acc = acc + d * d
                s = acc[0]
                for c in range(1, LANES):
                    s = s + acc[c]
                tot = jnp.where(lanes == t, s, tot)
            xi = lax.bitcast_convert_type(tot, jnp.int32)
            y = lax.bitcast_convert_type(
                magic - lax.shift_right_logical(xi, 1), jnp.float32)
            hx = half * tot
            for _ in range(3):
                y = y * (three_half - hx * y * y)
            out_v[pl.ds((k * GROUPS_PER_CHUNK + r) * LANES, LANES)] = tot * y
            return 0
        return group

    pending = fire(0, 0)
    for k in range(N_CHUNKS):
        nxt = fire(k + 1, (k + 1) % 2) if k + 1 < N_CHUNKS else None
        for c in pending:
            c.wait()
        lax.fori_loop(0, GROUPS_PER_CHUNK, make_group(k, k % 2), 0)
        pending = nxt

    pltpu.sync_copy(out_v, out_hbm.at[pl.ds(base, B_PER_W)])


TPOSE_C = 8192


def _tpose_body(in_ref, out_ref):
    eye = jnp.eye(DIM, dtype=jnp.float32)
    dn = (((0,), (0,)), ((), ()))
    for s in range(TPOSE_C // 1024):
        c0 = s * 1024
        t0 = lax.dot_general(in_ref[:, c0:c0 + 512], eye, dn,
                             preferred_element_type=jnp.float32)
        t1 = lax.dot_general(in_ref[:, c0 + 512:c0 + 1024], eye, dn,
                             preferred_element_type=jnp.float32)
        out_ref[pl.ds(s * 512, 512), 0:DIM] = t0
        out_ref[pl.ds(s * 512, 512), DIM:WIDE] = t1


def _tc_transpose(table_t, num_rows):
    grid = (num_rows + TPOSE_C - 1) // TPOSE_C
    return pl.pallas_call(
        _tpose_body,
        grid=(grid,),
        in_specs=[pl.BlockSpec((DIM, TPOSE_C), lambda i: (0, i))],
        out_specs=pl.BlockSpec((TPOSE_C // 2, WIDE), lambda i: (i, 0)),
        out_shape=jax.ShapeDtypeStruct(
            (grid * (TPOSE_C // 2), WIDE), jnp.float32),
    )(table_t)


@jax.jit
def kernel(triples, entity_table, relation_table):
    hidx = triples[:, 0].reshape(BATCH // CHUNK, CHUNK)
    ridx = triples[:, 1].reshape(BATCH // CHUNK, CHUNK)
    tidx = triples[:, 2].reshape(BATCH // CHUNK, CHUNK)
    ent2 = _tc_transpose(entity_table.T, entity_table.shape[0])
    rel2 = _tc_transpose(relation_table.T, relation_table.shape[0])

    run = functools.partial(
        pl.kernel,
        out_type=jax.ShapeDtypeStruct((BATCH,), jnp.float32),
        mesh=plsc.VectorSubcoreMesh(core_axis_name="c", subcore_axis_name="s"),
        scratch_types=[
            pltpu.VMEM((N_CHUNKS, CHUNK), jnp.int32),
            pltpu.VMEM((N_CHUNKS, CHUNK), jnp.int32),
            pltpu.VMEM((N_CHUNKS, CHUNK), jnp.int32),
            pltpu.VMEM((N_CHUNKS, CHUNK), jnp.int32),
            pltpu.VMEM((N_CHUNKS, CHUNK), jnp.int32),
            pltpu.VMEM((N_CHUNKS, CHUNK), jnp.int32),
            pltpu.VMEM((2, CHUNK, WIDE), jnp.float32),
            pltpu.VMEM((2, CHUNK, WIDE), jnp.float32),
            pltpu.VMEM((2, CHUNK, WIDE), jnp.float32),
            pltpu.VMEM((B_PER_W,), jnp.float32),
            pltpu.SemaphoreType.DMA,
            pltpu.SemaphoreType.DMA,
        ],
    )(_body)
    return run(ent2, rel2, hidx, ridx, tidx)


# speedup vs baseline: 2.5332x; 1.0024x over previous
import functools

import jax
import jax.numpy as jnp
from jax import lax
from jax.experimental import pallas as pl
from jax.experimental.pallas import tpu as pltpu
from jax.experimental.pallas import tpu_sc as plsc

BATCH = 16384
DIM = 64
WIDE = 128
LANES = 16
NUM_WORKERS = 32
B_PER_W = BATCH // NUM_WORKERS
CHUNK = 128
N_CHUNKS = B_PER_W // CHUNK
GROUPS_PER_CHUNK = CHUNK // LANES


def _body(ent_hbm, rel_hbm, hidx_hbm, ridx_hbm, tidx_hbm, out_hbm,
          hidx_v, ridx_v, tidx_v, hhalf_v, rhalf_v, thalf_v,
          hrows_v, rrows_v, trows_v, out_v, *sems):
    wid = lax.axis_index("s") * 2 + lax.axis_index("c")
    row0 = wid * N_CHUNKS
    base = wid * B_PER_W

    pltpu.sync_copy(hidx_hbm.at[pl.ds(row0, N_CHUNKS)], hidx_v)
    pltpu.sync_copy(ridx_hbm.at[pl.ds(row0, N_CHUNKS)], ridx_v)
    pltpu.sync_copy(tidx_hbm.at[pl.ds(row0, N_CHUNKS)], tidx_v)

    m4095 = jnp.full((LANES,), 4095, jnp.int32)
    for src, dst in ((hidx_v, hhalf_v), (ridx_v, rhalf_v), (tidx_v, thalf_v)):
        for k in range(N_CHUNKS):
            for v in range(CHUNK // LANES):
                sl = pl.ds(v * LANES, LANES)
                e = src[k, sl]
                dst[k, sl] = (
                    lax.shift_left(lax.shift_right_logical(e, 13), 12)
                    + (e & m4095))

    lanes = lax.iota(jnp.int32, LANES)
    zero = jnp.zeros((LANES,), jnp.float32)
    half = jnp.full((LANES,), 0.5, jnp.float32)
    three_half = jnp.full((LANES,), 1.5, jnp.float32)
    magic = jnp.full((LANES,), 0x5F3759DF, jnp.int32)
    six = jnp.int32(6)
    one = jnp.int32(1)

    def fire(k, slot):
        s = sems[slot]
        return (
            pltpu.async_copy(ent_hbm.at[hhalf_v.at[k]], hrows_v.at[slot], s),
            pltpu.async_copy(rel_hbm.at[rhalf_v.at[k]], rrows_v.at[slot], s),
            pltpu.async_copy(ent_hbm.at[thalf_v.at[k]], trows_v.at[slot], s),
        )

    def make_group(k, slot):
        def group(r, _):
            sl16 = pl.ds(r * LANES, LANES)
            hv = hidx_v[k, sl16]
            rv = ridx_v[k, sl16]
            tv = tidx_v[k, sl16]
            tot = zero
            for t in range(LANES):
                i = r * LANES + t
                ho = lax.shift_left(
                    lax.shift_right_logical(hv[t], 12) & one, six)
                ro = lax.shift_left(
                    lax.shift_right_logical(rv[t], 12) & one, six)
                to = lax.shift_left(
                    lax.shift_right_logical(tv[t], 12) & one, six)
                acc = zero
                for j in range(DIM // LANES):
                    o = j * LANES
                    d = (hrows_v[slot, i, pl.ds(ho + o, LANES)]
                         + rrows_v[slot, i, pl.ds(ro + o, LANES)]
                         - trows_v[slot, i, pl.ds(to + o, LANES)])
                    acc = acc + d * d
                s = acc[0]
                for c in range(1, LANES):
                    s = s + acc[c]
                tot = jnp.where(lanes == t, s, tot)
            xi = lax.bitcast_convert_type(tot, jnp.int32)
            y = lax.bitcast_convert_type(
                magic - lax.shift_right_logical(xi, 1), jnp.float32)
            hx = half * tot
            for _ in range(3):
                y = y * (three_half - hx * y * y)
            out_v[pl.ds((k * GROUPS_PER_CHUNK + r) * LANES, LANES)] = tot * y
            return 0
        return group

    pending = fire(0, 0)
    for k in range(N_CHUNKS):
        nxt = fire(k + 1, (k + 1) % 2) if k + 1 < N_CHUNKS else None
        for c in pending:
            c.wait()
        lax.fori_loop(0, GROUPS_PER_CHUNK, make_group(k, k % 2), 0)
        pending = nxt

    pltpu.sync_copy(out_v, out_hbm.at[pl.ds(base, B_PER_W)])


TPOSE_C = 8192


def _tpose_body(in_ref, out_ref):
    eye = jnp.eye(DIM, dtype=jnp.float32)
    dn = (((0,), (0,)), ((), ()))
    half = TPOSE_C // 2
    out_ref[:, 0:DIM] = lax.dot_general(
        in_ref[:, 0:half], eye, dn, preferred_element_type=jnp.float32)
    out_ref[:, DIM:WIDE] = lax.dot_general(
        in_ref[:, half:TPOSE_C], eye, dn,
        preferred_element_type=jnp.float32)


def _tc_transpose(table_t, num_rows):
    grid = (num_rows + TPOSE_C - 1) // TPOSE_C
    return pl.pallas_call(
        _tpose_body,
        grid=(grid,),
        in_specs=[pl.BlockSpec((DIM, TPOSE_C), lambda i: (0, i))],
        out_specs=pl.BlockSpec((TPOSE_C // 2, WIDE), lambda i: (i, 0)),
        out_shape=jax.ShapeDtypeStruct(
            (grid * (TPOSE_C // 2), WIDE), jnp.float32),
    )(table_t)


@jax.jit
def kernel(triples, entity_table, relation_table):
    hidx = triples[:, 0].reshape(BATCH // CHUNK, CHUNK)
    ridx = triples[:, 1].reshape(BATCH // CHUNK, CHUNK)
    tidx = triples[:, 2].reshape(BATCH // CHUNK, CHUNK)
    ent2 = _tc_transpose(entity_table.T, entity_table.shape[0])
    rel2 = _tc_transpose(relation_table.T, relation_table.shape[0])

    run = functools.partial(
        pl.kernel,
        out_type=jax.ShapeDtypeStruct((BATCH,), jnp.float32),
        mesh=plsc.VectorSubcoreMesh(core_axis_name="c", subcore_axis_name="s"),
        scratch_types=[
            pltpu.VMEM((N_CHUNKS, CHUNK), jnp.int32),
            pltpu.VMEM((N_CHUNKS, CHUNK), jnp.int32),
            pltpu.VMEM((N_CHUNKS, CHUNK), jnp.int32),
            pltpu.VMEM((N_CHUNKS, CHUNK), jnp.int32),
            pltpu.VMEM((N_CHUNKS, CHUNK), jnp.int32),
            pltpu.VMEM((N_CHUNKS, CHUNK), jnp.int32),
            pltpu.VMEM((2, CHUNK, WIDE), jnp.float32),
            pltpu.VMEM((2, CHUNK, WIDE), jnp.float32),
            pltpu.VMEM((2, CHUNK, WIDE), jnp.float32),
            pltpu.VMEM((B_PER_W,), jnp.float32),
            pltpu.SemaphoreType.DMA,
            pltpu.SemaphoreType.DMA,
        ],
    )(_body)
    return run(ent2, rel2, hidx, ridx, tidx)


# speedup vs baseline: 2.8736x; 1.1344x over previous
import functools

import jax
import jax.numpy as jnp
from jax import lax
from jax.experimental import pallas as pl
from jax.experimental.pallas import tpu as pltpu
from jax.experimental.pallas import tpu_sc as plsc

BATCH = 16384
DIM = 64
WIDE = 128
LANES = 16
NUM_WORKERS = 32
B_PER_W = BATCH // NUM_WORKERS
CHUNK = 128
N_CHUNKS = B_PER_W // CHUNK
GROUPS_PER_CHUNK = CHUNK // LANES


def _body(ent_hbm, rel_hbm, hidx_hbm, ridx_hbm, tidx_hbm, out_hbm,
          hidx_v, ridx_v, tidx_v, hhalf_v, rhalf_v, thalf_v,
          hrows_v, rrows_v, trows_v, out_v, *sems):
    wid = lax.axis_index("s") * 2 + lax.axis_index("c")
    row0 = wid * N_CHUNKS
    base = wid * B_PER_W

    pltpu.sync_copy(hidx_hbm.at[pl.ds(row0, N_CHUNKS)], hidx_v)
    pltpu.sync_copy(ridx_hbm.at[pl.ds(row0, N_CHUNKS)], ridx_v)
    pltpu.sync_copy(tidx_hbm.at[pl.ds(row0, N_CHUNKS)], tidx_v)

    mhalf = jnp.full((LANES,), 8191, jnp.int32)
    for src, dst in ((hidx_v, hhalf_v), (ridx_v, rhalf_v), (tidx_v, thalf_v)):
        for k in range(N_CHUNKS):
            for v in range(CHUNK // LANES):
                sl = pl.ds(v * LANES, LANES)
                e = src[k, sl]
                dst[k, sl] = (
                    lax.shift_left(lax.shift_right_logical(e, 14), 13)
                    + (e & mhalf))

    lanes = lax.iota(jnp.int32, LANES)
    zero = jnp.zeros((LANES,), jnp.float32)
    half = jnp.full((LANES,), 0.5, jnp.float32)
    three_half = jnp.full((LANES,), 1.5, jnp.float32)
    magic = jnp.full((LANES,), 0x5F3759DF, jnp.int32)
    six = jnp.int32(6)
    one = jnp.int32(1)

    def fire(k, slot):
        s = sems[slot]
        return (
            pltpu.async_copy(ent_hbm.at[hhalf_v.at[k]], hrows_v.at[slot], s),
            pltpu.async_copy(rel_hbm.at[rhalf_v.at[k]], rrows_v.at[slot], s),
            pltpu.async_copy(ent_hbm.at[thalf_v.at[k]], trows_v.at[slot], s),
        )

    def make_group(k, slot):
        def group(r, _):
            sl16 = pl.ds(r * LANES, LANES)
            hv = hidx_v[k, sl16]
            rv = ridx_v[k, sl16]
            tv = tidx_v[k, sl16]
            tot = zero
            for t in range(LANES):
                i = r * LANES + t
                ho = lax.shift_left(
                    lax.shift_right_logical(hv[t], 13) & one, six)
                ro = lax.shift_left(
                    lax.shift_right_logical(rv[t], 13) & one, six)
                to = lax.shift_left(
                    lax.shift_right_logical(tv[t], 13) & one, six)
                acc = zero
                for j in range(DIM // LANES):
                    o = j * LANES
                    d = (hrows_v[slot, i, pl.ds(ho + o, LANES)]
                         + rrows_v[slot, i, pl.ds(ro + o, LANES)]
                         - trows_v[slot, i, pl.ds(to + o, LANES)])
                    acc = acc + d * d
                s = acc[0]
                for c in range(1, LANES):
                    s = s + acc[c]
                tot = jnp.where(lanes == t, s, tot)
            xi = lax.bitcast_convert_type(tot, jnp.int32)
            y = lax.bitcast_convert_type(
                magic - lax.shift_right_logical(xi, 1), jnp.float32)
            hx = half * tot
            for _ in range(3):
                y = y * (three_half - hx * y * y)
            out_v[pl.ds((k * GROUPS_PER_CHUNK + r) * LANES, LANES)] = tot * y
            return 0
        return group

    pending = fire(0, 0)
    for k in range(N_CHUNKS):
        nxt = fire(k + 1, (k + 1) % 2) if k + 1 < N_CHUNKS else None
        for c in pending:
            c.wait()
        lax.fori_loop(0, GROUPS_PER_CHUNK, make_group(k, k % 2), 0)
        pending = nxt

    pltpu.sync_copy(out_v, out_hbm.at[pl.ds(base, B_PER_W)])


TPOSE_C = 16384


def _tpose_body(in_ref, out_ref):
    eye = jnp.eye(DIM, dtype=jnp.float32)
    dn = (((0,), (0,)), ((), ()))
    half = TPOSE_C // 2
    out_ref[:, 0:DIM] = lax.dot_general(
        in_ref[:, 0:half], eye, dn, preferred_element_type=jnp.float32)
    out_ref[:, DIM:WIDE] = lax.dot_general(
        in_ref[:, half:TPOSE_C], eye, dn,
        preferred_element_type=jnp.float32)


def _tc_transpose(table_t, num_rows):
    grid = (num_rows + TPOSE_C - 1) // TPOSE_C
    return pl.pallas_call(
        _tpose_body,
        grid=(grid,),
        in_specs=[pl.BlockSpec((DIM, TPOSE_C), lambda i: (0, i))],
        out_specs=pl.BlockSpec((TPOSE_C // 2, WIDE), lambda i: (i, 0)),
        out_shape=jax.ShapeDtypeStruct(
            (grid * (TPOSE_C // 2), WIDE), jnp.float32),
    )(table_t)


@jax.jit
def kernel(triples, entity_table, relation_table):
    hidx = triples[:, 0].reshape(BATCH // CHUNK, CHUNK)
    ridx = triples[:, 1].reshape(BATCH // CHUNK, CHUNK)
    tidx = triples[:, 2].reshape(BATCH // CHUNK, CHUNK)
    ent2 = _tc_transpose(entity_table.T, entity_table.shape[0])
    rel2 = _tc_transpose(relation_table.T, relation_table.shape[0])

    run = functools.partial(
        pl.kernel,
        out_type=jax.ShapeDtypeStruct((BATCH,), jnp.float32),
        mesh=plsc.VectorSubcoreMesh(core_axis_name="c", subcore_axis_name="s"),
        scratch_types=[
            pltpu.VMEM((N_CHUNKS, CHUNK), jnp.int32),
            pltpu.VMEM((N_CHUNKS, CHUNK), jnp.int32),
            pltpu.VMEM((N_CHUNKS, CHUNK), jnp.int32),
            pltpu.VMEM((N_CHUNKS, CHUNK), jnp.int32),
            pltpu.VMEM((N_CHUNKS, CHUNK), jnp.int32),
            pltpu.VMEM((N_CHUNKS, CHUNK), jnp.int32),
            pltpu.VMEM((2, CHUNK, WIDE), jnp.float32),
            pltpu.VMEM((2, CHUNK, WIDE), jnp.float32),
            pltpu.VMEM((2, CHUNK, WIDE), jnp.float32),
            pltpu.VMEM((B_PER_W,), jnp.float32),
            pltpu.SemaphoreType.DMA,
            pltpu.SemaphoreType.DMA,
        ],
    )(_body)
    return run(ent2, rel2, hidx, ridx, tidx)


# speedup vs baseline: 3.0437x; 1.0592x over previous
import functools

import jax
import jax.numpy as jnp
from jax import lax
from jax.experimental import pallas as pl
from jax.experimental.pallas import tpu as pltpu
from jax.experimental.pallas import tpu_sc as plsc

BATCH = 16384
DIM = 64
WIDE = 128
LANES = 16
NUM_WORKERS = 32
B_PER_W = BATCH // NUM_WORKERS
CHUNK = 128
N_CHUNKS = B_PER_W // CHUNK
GROUPS_PER_CHUNK = CHUNK // LANES


def _body(ent_hbm, rel_hbm, hidx_hbm, ridx_hbm, tidx_hbm, out_hbm,
          hidx_v, ridx_v, tidx_v, hhalf_v, rhalf_v, thalf_v,
          hrows_v, rrows_v, trows_v, out_v, *sems):
    wid = lax.axis_index("s") * 2 + lax.axis_index("c")
    row0 = wid * N_CHUNKS
    base = wid * B_PER_W

    pltpu.sync_copy(hidx_hbm.at[pl.ds(row0, N_CHUNKS)], hidx_v)
    pltpu.sync_copy(ridx_hbm.at[pl.ds(row0, N_CHUNKS)], ridx_v)
    pltpu.sync_copy(tidx_hbm.at[pl.ds(row0, N_CHUNKS)], tidx_v)

    mhalf = jnp.full((LANES,), 16383, jnp.int32)
    for src, dst in ((hidx_v, hhalf_v), (ridx_v, rhalf_v), (tidx_v, thalf_v)):
        for k in range(N_CHUNKS):
            for v in range(CHUNK // LANES):
                sl = pl.ds(v * LANES, LANES)
                e = src[k, sl]
                dst[k, sl] = (
                    lax.shift_left(lax.shift_right_logical(e, 15), 14)
                    + (e & mhalf))

    lanes = lax.iota(jnp.int32, LANES)
    zero = jnp.zeros((LANES,), jnp.float32)
    half = jnp.full((LANES,), 0.5, jnp.float32)
    three_half = jnp.full((LANES,), 1.5, jnp.float32)
    magic = jnp.full((LANES,), 0x5F3759DF, jnp.int32)
    six = jnp.int32(6)
    one = jnp.int32(1)

    def fire(k, slot):
        s = sems[slot]
        return (
            pltpu.async_copy(ent_hbm.at[hhalf_v.at[k]], hrows_v.at[slot], s),
            pltpu.async_copy(rel_hbm.at[rhalf_v.at[k]], rrows_v.at[slot], s),
            pltpu.async_copy(ent_hbm.at[thalf_v.at[k]], trows_v.at[slot], s),
        )

    def make_group(k, slot):
        def group(r, _):
            sl16 = pl.ds(r * LANES, LANES)
            hv = hidx_v[k, sl16]
            rv = ridx_v[k, sl16]
            tv = tidx_v[k, sl16]
            tot = zero
            for t in range(LANES):
                i = r * LANES + t
                ho = lax.shift_left(
                    lax.shift_right_logical(hv[t], 14) & one, six)
                ro = lax.shift_left(
                    lax.shift_right_logical(rv[t], 14) & one, six)
                to = lax.shift_left(
                    lax.shift_right_logical(tv[t], 14) & one, six)
                acc = zero
                for j in range(DIM // LANES):
                    o = j * LANES
                    d = (hrows_v[slot, i, pl.ds(ho + o, LANES)]
                         + rrows_v[slot, i, pl.ds(ro + o, LANES)]
                         - trows_v[slot, i, pl.ds(to + o, LANES)])
                    acc = acc + d * d
                s = acc[0]
                for c in range(1, LANES):
                    s = s + acc[c]
                tot = jnp.where(lanes == t, s, tot)
            xi = lax.bitcast_convert_type(tot, jnp.int32)
            y = lax.bitcast_convert_type(
                magic - lax.shift_right_logical(xi, 1), jnp.float32)
            hx = half * tot
            for _ in range(3):
                y = y * (three_half - hx * y * y)
            out_v[pl.ds((k * GROUPS_PER_CHUNK + r) * LANES, LANES)] = tot * y
            return 0
        return group

    pending = fire(0, 0)
    for k in range(N_CHUNKS):
        nxt = fire(k + 1, (k + 1) % 2) if k + 1 < N_CHUNKS else None
        for c in pending:
            c.wait()
        lax.fori_loop(0, GROUPS_PER_CHUNK, make_group(k, k % 2), 0)
        pending = nxt

    pltpu.sync_copy(out_v, out_hbm.at[pl.ds(base, B_PER_W)])


TPOSE_C = 32768


def _tpose_body(in_ref, out_ref):
    eye = jnp.eye(DIM, dtype=jnp.float32)
    dn = (((0,), (0,)), ((), ()))
    half = TPOSE_C // 2
    out_ref[:, 0:DIM] = lax.dot_general(
        in_ref[:, 0:half], eye, dn, preferred_element_type=jnp.float32)
    out_ref[:, DIM:WIDE] = lax.dot_general(
        in_ref[:, half:TPOSE_C], eye, dn,
        preferred_element_type=jnp.float32)


def _tc_transpose(table_t, num_rows):
    grid = (num_rows + TPOSE_C - 1) // TPOSE_C
    return pl.pallas_call(
        _tpose_body,
        grid=(grid,),
        in_specs=[pl.BlockSpec((DIM, TPOSE_C), lambda i: (0, i))],
        out_specs=pl.BlockSpec((TPOSE_C // 2, WIDE), lambda i: (i, 0)),
        out_shape=jax.ShapeDtypeStruct(
            (grid * (TPOSE_C // 2), WIDE), jnp.float32),
    )(table_t)


@jax.jit
def kernel(triples, entity_table, relation_table):
    hidx = triples[:, 0].reshape(BATCH // CHUNK, CHUNK)
    ridx = triples[:, 1].reshape(BATCH // CHUNK, CHUNK)
    tidx = triples[:, 2].reshape(BATCH // CHUNK, CHUNK)
    ent2 = _tc_transpose(entity_table.T, entity_table.shape[0])
    rel2 = _tc_transpose(relation_table.T, relation_table.shape[0])

    run = functools.partial(
        pl.kernel,
        out_type=jax.ShapeDtypeStruct((BATCH,), jnp.float32),
        mesh=plsc.VectorSubcoreMesh(core_axis_name="c", subcore_axis_name="s"),
        scratch_types=[
            pltpu.VMEM((N_CHUNKS, CHUNK), jnp.int32),
            pltpu.VMEM((N_CHUNKS, CHUNK), jnp.int32),
            pltpu.VMEM((N_CHUNKS, CHUNK), jnp.int32),
            pltpu.VMEM((N_CHUNKS, CHUNK), jnp.int32),
            pltpu.VMEM((N_CHUNKS, CHUNK), jnp.int32),
            pltpu.VMEM((N_CHUNKS, CHUNK), jnp.int32),
            pltpu.VMEM((2, CHUNK, WIDE), jnp.float32),
            pltpu.VMEM((2, CHUNK, WIDE), jnp.float32),
            pltpu.VMEM((2, CHUNK, WIDE), jnp.float32),
            pltpu.VMEM((B_PER_W,), jnp.float32),
            pltpu.SemaphoreType.DMA,
            pltpu.SemaphoreType.DMA,
        ],
    )(_body)
    return run(ent2, rel2, hidx, ridx, tidx)
